# baseline (speedup 1.0000x reference)
import functools

import jax
import jax.numpy as jnp
from jax import lax
from jax.experimental import pallas as pl
from jax.experimental.pallas import tpu as pltpu
from jax.experimental.pallas import tpu_sc as plsc

N = 10000
E = 320000
D = 128

NC = 2
NS = 16
EPS = E // (NC * NS)
C = 80
CHF = EPS // C
RPS = 640
RLAST = N - 15 * RPS
RCH = 80
CW = 16
NB = 5
GIF = 2
SIF = 3


def _sc_agg_body(do_cnt, *refs):
    if do_cnt:
        (xs, srcb, dstb, zrow, zcnt, orow, agg_o, cnt_o,
         srcv, dstv, rows, zbuf, onesb, aggsh, cntsh, sem) = refs
    else:
        (xs, srcb, dstb, zrow, agg_o,
         srcv, dstv, rows, zbuf, aggsh, sem) = refs

    cid = lax.axis_index("c")
    sid = lax.axis_index("s")

    pltpu.sync_copy(srcb.at[cid, sid], srcv)
    pltpu.sync_copy(dstb.at[cid, sid], dstv)

    pltpu.sync_copy(zrow, zbuf)
    if do_cnt:
        pltpu.sync_copy(orow, onesb)
    r0 = sid * RPS

    @pl.when(sid < NS - 1)
    def _():
        for k in range(RPS // RCH):
            pltpu.sync_copy(zbuf, aggsh.at[pl.ds(r0 + k * RCH, RCH)])
        if do_cnt:
            pltpu.sync_copy(zcnt, cntsh.at[pl.ds(r0, RPS)])

    @pl.when(sid == NS - 1)
    def _():
        for k in range(RLAST // RCH):
            pltpu.sync_copy(zbuf, aggsh.at[pl.ds(r0 + k * RCH, RCH)])
        if do_cnt:
            pltpu.sync_copy(zcnt.at[pl.ds(0, RLAST)], cntsh.at[pl.ds(r0, RLAST)])

    plsc.subcore_barrier()

    semg, sems, semc = sem
    for b in range(GIF):
        pltpu.async_copy(xs.at[srcv.at[b]], rows.at[b], semg)

    def step(gN, carry):
        for b in range(NB):
            g = gN * NB + b
            pltpu.make_async_copy(xs.at[srcv.at[g]], rows.at[b], semg).wait()
            pltpu.async_copy(rows.at[b], aggsh.at[dstv.at[g]], sems, add=True)
            if do_cnt:
                pltpu.async_copy(onesb, cntsh.at[dstv.at[g]], semc, add=True)

            @pl.when(g >= SIF)
            def _():
                pltpu.make_async_copy(rows.at[(b + NB - SIF) % NB],
                                      aggsh.at[dstv.at[g - SIF]], sems).wait()

            @pl.when(g + GIF < CHF)
            def _():
                pltpu.async_copy(xs.at[srcv.at[g + GIF]],
                                 rows.at[(b + GIF) % NB], semg)
        return carry

    lax.fori_loop(0, CHF // NB, step, 0)
    for t in range(SIF):
        g = CHF - SIF + t
        pltpu.make_async_copy(rows.at[g % NB],
                              aggsh.at[dstv.at[g]], sems).wait()
    if do_cnt:
        def drain(i, carry):
            pltpu.make_async_copy(onesb, cntsh.at[dstv.at[i]], semc).wait()
            return carry
        lax.fori_loop(0, CHF, drain, 0)

    plsc.subcore_barrier()

    @pl.when(sid < NS - 1)
    def _():
        pltpu.sync_copy(aggsh.at[pl.ds(r0, RPS)],
                        agg_o.at[cid, pl.ds(r0, RPS)])
        if do_cnt:
            pltpu.sync_copy(cntsh.at[pl.ds(r0, RPS)],
                            cnt_o.at[cid, pl.ds(r0, RPS)])

    @pl.when(sid == NS - 1)
    def _():
        pltpu.sync_copy(aggsh.at[pl.ds(r0, RLAST)],
                        agg_o.at[cid, pl.ds(r0, RLAST)])
        if do_cnt:
            pltpu.sync_copy(cntsh.at[pl.ds(r0, RLAST)],
                            cnt_o.at[cid, pl.ds(r0, RLAST)])


def _make_sc_agg(do_cnt):
    mesh = plsc.VectorSubcoreMesh(core_axis_name="c", subcore_axis_name="s",
                                  num_cores=NC, num_subcores=NS)
    out_type = [jax.ShapeDtypeStruct((NC, N, D), jnp.bfloat16)]
    scratch = [
        pltpu.VMEM((CHF, C), jnp.int32),
        pltpu.VMEM((CHF, C), jnp.int32),
        pltpu.VMEM((NB, C, D), jnp.bfloat16),
        pltpu.VMEM((RCH, D), jnp.bfloat16),
    ]
    if do_cnt:
        out_type.append(jax.ShapeDtypeStruct((NC, N, CW), jnp.float32))
        scratch.append(pltpu.VMEM((C, CW), jnp.float32))
    scratch.append(pltpu.VMEM_SHARED((N, D), jnp.bfloat16))
    if do_cnt:
        scratch.append(pltpu.VMEM_SHARED((N, CW), jnp.float32))
    scratch.append((pltpu.SemaphoreType.DMA, pltpu.SemaphoreType.DMA,
                    pltpu.SemaphoreType.DMA))
    return pl.kernel(functools.partial(_sc_agg_body, do_cnt),
                     out_type=out_type, mesh=mesh, scratch_types=scratch,
                     compiler_params=pltpu.CompilerParams(
                         use_tc_tiling_on_sc=False))


_sc_agg_cnt = _make_sc_agg(True)
_sc_agg = _make_sc_agg(False)

BN = 1000


def _root_body(h_ref, wr_ref, bl_ref, o_ref):
    y = jnp.dot(h_ref[...], wr_ref[...], preferred_element_type=jnp.float32)
    o_ref[...] = y + h_ref[...] + bl_ref[...]


_root = pl.pallas_call(
    _root_body,
    grid=(N // BN,),
    in_specs=[
        pl.BlockSpec((BN, D), lambda i: (i, 0)),
        pl.BlockSpec((D, D), lambda i: (0, 0)),
        pl.BlockSpec((1, D), lambda i: (0, 0)),
    ],
    out_specs=pl.BlockSpec((BN, D), lambda i: (i, 0)),
    out_shape=jax.ShapeDtypeStruct((N, D), jnp.float32),
)


def _dense_body(relu, agg_ref, cnt_ref, p_ref, wl_ref, o_ref, ob_ref=None):
    cnt = cnt_ref[0, :, 0] + cnt_ref[1, :, 0]
    agg = agg_ref[0].astype(jnp.float32) + agg_ref[1].astype(jnp.float32)
    mean = agg / jnp.maximum(cnt, 1.0)[:, None]
    y = jnp.dot(mean, wl_ref[...], preferred_element_type=jnp.float32)
    y = y + p_ref[...]
    if relu:
        y = jnp.maximum(y, 0.0)
    o_ref[...] = y
    if ob_ref is not None:
        ob_ref[...] = y.astype(jnp.bfloat16)


def _make_dense(relu, emit_bf16):
    out_specs = [pl.BlockSpec((BN, D), lambda i: (i, 0))]
    out_shape = [jax.ShapeDtypeStruct((N, D), jnp.float32)]
    if emit_bf16:
        out_specs.append(pl.BlockSpec((BN, D), lambda i: (i, 0)))
        out_shape.append(jax.ShapeDtypeStruct((N, D), jnp.bfloat16))
    return pl.pallas_call(
        functools.partial(_dense_body, relu),
        grid=(N // BN,),
        in_specs=[
            pl.BlockSpec((NC, BN, D), lambda i: (0, i, 0)),
            pl.BlockSpec((NC, BN, CW), lambda i: (0, i, 0)),
            pl.BlockSpec((BN, D), lambda i: (i, 0)),
            pl.BlockSpec((D, D), lambda i: (0, 0)),
        ],
        out_specs=out_specs,
        out_shape=out_shape,
    )


_dense_relu = _make_dense(True, True)
_dense_plain = _make_dense(False, False)


def kernel(x, edge_index, Wl1, bl1, Wr1, Wl2, bl2, Wr2):
    srcb = edge_index[0].reshape(NC, NS, CHF, C)
    dstb = edge_index[1].reshape(NC, NS, CHF, C)
    zrow = jnp.zeros((RCH, D), jnp.bfloat16)
    zcnt = jnp.zeros((RPS, CW), jnp.float32)
    orow = jnp.ones((C, CW), jnp.float32)

    xb = x.astype(jnp.bfloat16)
    agg1, cntp = _sc_agg_cnt(xb, srcb, dstb, zrow, zcnt, orow)
    p1 = _root(x, Wr1, bl1.reshape(1, D))
    h, hb = _dense_relu(agg1, cntp, p1, Wl1)
    (agg2,) = _sc_agg(hb, srcb, dstb, zrow)
    p2 = _root(h, Wr2, bl2.reshape(1, D))
    (out,) = _dense_plain(agg2, cntp, p2, Wl2)
    return out


# speedup vs baseline: 1.1154x; 1.1154x over previous
import functools

import jax
import jax.numpy as jnp
from jax import lax
from jax.experimental import pallas as pl
from jax.experimental.pallas import tpu as pltpu
from jax.experimental.pallas import tpu_sc as plsc

N = 10000
E = 320000
D = 128

NC = 2
NS = 16
EPS = E // (NC * NS)
C = 80
CHF = EPS // C
RPS = 640
RLAST = N - 15 * RPS
RCH = 80
CW = 16
NB = 5
GIF = 3
SIF = 2


def _sc_agg_body(do_cnt, *refs):
    if do_cnt:
        (xs, srcb, dstb, zrow, zcnt, orow, agg_o, cnt_o,
         srcv, dstv, rows, zbuf, onesb, aggsh, cntsh, sem) = refs
    else:
        (xs, srcb, dstb, zrow, agg_o,
         srcv, dstv, rows, zbuf, aggsh, sem) = refs

    cid = lax.axis_index("c")
    sid = lax.axis_index("s")

    pltpu.sync_copy(srcb.at[cid, sid], srcv)
    pltpu.sync_copy(dstb.at[cid, sid], dstv)

    pltpu.sync_copy(zrow, zbuf)
    if do_cnt:
        pltpu.sync_copy(orow, onesb)
    r0 = sid * RPS

    @pl.when(sid < NS - 1)
    def _():
        for k in range(RPS // RCH):
            pltpu.sync_copy(zbuf, aggsh.at[pl.ds(r0 + k * RCH, RCH)])
        if do_cnt:
            pltpu.sync_copy(zcnt, cntsh.at[pl.ds(r0, RPS)])

    @pl.when(sid == NS - 1)
    def _():
        for k in range(RLAST // RCH):
            pltpu.sync_copy(zbuf, aggsh.at[pl.ds(r0 + k * RCH, RCH)])
        if do_cnt:
            pltpu.sync_copy(zcnt.at[pl.ds(0, RLAST)], cntsh.at[pl.ds(r0, RLAST)])

    plsc.subcore_barrier()

    semg, sems, semc = sem
    for b in range(GIF):
        pltpu.async_copy(xs.at[srcv.at[b]], rows.at[b], semg)

    def step(gN, carry):
        for b in range(NB):
            g = gN * NB + b
            pltpu.make_async_copy(xs.at[srcv.at[g]], rows.at[b], semg).wait()
            pltpu.async_copy(rows.at[b], aggsh.at[dstv.at[g]], sems, add=True)
            if do_cnt:
                pltpu.async_copy(onesb, cntsh.at[dstv.at[g]], semc, add=True)

            @pl.when(g >= SIF)
            def _():
                pltpu.make_async_copy(rows.at[(b + NB - SIF) % NB],
                                      aggsh.at[dstv.at[g - SIF]], sems).wait()

            @pl.when(g + GIF < CHF)
            def _():
                pltpu.async_copy(xs.at[srcv.at[g + GIF]],
                                 rows.at[(b + GIF) % NB], semg)
        return carry

    lax.fori_loop(0, CHF // NB, step, 0)
    for t in range(SIF):
        g = CHF - SIF + t
        pltpu.make_async_copy(rows.at[g % NB],
                              aggsh.at[dstv.at[g]], sems).wait()
    if do_cnt:
        def drain(i, carry):
            pltpu.make_async_copy(onesb, cntsh.at[dstv.at[i]], semc).wait()
            return carry
        lax.fori_loop(0, CHF, drain, 0)

    plsc.subcore_barrier()

    @pl.when(sid < NS - 1)
    def _():
        pltpu.sync_copy(aggsh.at[pl.ds(r0, RPS)],
                        agg_o.at[cid, pl.ds(r0, RPS)])
        if do_cnt:
            pltpu.sync_copy(cntsh.at[pl.ds(r0, RPS)],
                            cnt_o.at[cid, pl.ds(r0, RPS)])

    @pl.when(sid == NS - 1)
    def _():
        pltpu.sync_copy(aggsh.at[pl.ds(r0, RLAST)],
                        agg_o.at[cid, pl.ds(r0, RLAST)])
        if do_cnt:
            pltpu.sync_copy(cntsh.at[pl.ds(r0, RLAST)],
                            cnt_o.at[cid, pl.ds(r0, RLAST)])


def _make_sc_agg(do_cnt):
    mesh = plsc.VectorSubcoreMesh(core_axis_name="c", subcore_axis_name="s",
                                  num_cores=NC, num_subcores=NS)
    out_type = [jax.ShapeDtypeStruct((NC, N, D), jnp.bfloat16)]
    scratch = [
        pltpu.VMEM((CHF, C), jnp.int32),
        pltpu.VMEM((CHF, C), jnp.int32),
        pltpu.VMEM((NB, C, D), jnp.bfloat16),
        pltpu.VMEM((RCH, D), jnp.bfloat16),
    ]
    if do_cnt:
        out_type.append(jax.ShapeDtypeStruct((NC, N, CW), jnp.float32))
        scratch.append(pltpu.VMEM((C, CW), jnp.float32))
    scratch.append(pltpu.VMEM_SHARED((N, D), jnp.bfloat16))
    if do_cnt:
        scratch.append(pltpu.VMEM_SHARED((N, CW), jnp.float32))
    scratch.append((pltpu.SemaphoreType.DMA, pltpu.SemaphoreType.DMA,
                    pltpu.SemaphoreType.DMA))
    return pl.kernel(functools.partial(_sc_agg_body, do_cnt),
                     out_type=out_type, mesh=mesh, scratch_types=scratch,
                     compiler_params=pltpu.CompilerParams(
                         use_tc_tiling_on_sc=False))


_sc_agg_cnt = _make_sc_agg(True)
_sc_agg = _make_sc_agg(False)

BN = 1000


def _dense_body(relu, agg_ref, cnt_ref, h_ref, wl_ref, bl_ref, wr_ref, o_ref,
                ob_ref=None):
    cnt = cnt_ref[0, :, 0] + cnt_ref[1, :, 0]
    agg = agg_ref[0].astype(jnp.float32) + agg_ref[1].astype(jnp.float32)
    mean = agg / jnp.maximum(cnt, 1.0)[:, None]
    y = jnp.dot(mean, wl_ref[...], preferred_element_type=jnp.float32)
    y = y + jnp.dot(h_ref[...], wr_ref[...], preferred_element_type=jnp.float32)
    y = y + h_ref[...] + bl_ref[...]
    if relu:
        y = jnp.maximum(y, 0.0)
    o_ref[...] = y
    if ob_ref is not None:
        ob_ref[...] = y.astype(jnp.bfloat16)


def _make_dense(relu, emit_bf16):
    out_specs = [pl.BlockSpec((BN, D), lambda i: (i, 0))]
    out_shape = [jax.ShapeDtypeStruct((N, D), jnp.float32)]
    if emit_bf16:
        out_specs.append(pl.BlockSpec((BN, D), lambda i: (i, 0)))
        out_shape.append(jax.ShapeDtypeStruct((N, D), jnp.bfloat16))
    return pl.pallas_call(
        functools.partial(_dense_body, relu),
        grid=(N // BN,),
        in_specs=[
            pl.BlockSpec((NC, BN, D), lambda i: (0, i, 0)),
            pl.BlockSpec((NC, BN, CW), lambda i: (0, i, 0)),
            pl.BlockSpec((BN, D), lambda i: (i, 0)),
            pl.BlockSpec((D, D), lambda i: (0, 0)),
            pl.BlockSpec((1, D), lambda i: (0, 0)),
            pl.BlockSpec((D, D), lambda i: (0, 0)),
        ],
        out_specs=out_specs,
        out_shape=out_shape,
    )


_dense_relu = _make_dense(True, True)
_dense_plain = _make_dense(False, False)


def kernel(x, edge_index, Wl1, bl1, Wr1, Wl2, bl2, Wr2):
    srcb = edge_index[0].reshape(NC, NS, CHF, C)
    dstb = edge_index[1].reshape(NC, NS, CHF, C)
    zrow = jnp.zeros((RCH, D), jnp.bfloat16)
    zcnt = jnp.zeros((RPS, CW), jnp.float32)
    orow = jnp.ones((C, CW), jnp.float32)

    xb = x.astype(jnp.bfloat16)
    agg1, cntp = _sc_agg_cnt(xb, srcb, dstb, zrow, zcnt, orow)
    h, hb = _dense_relu(agg1, cntp, x, Wl1, bl1.reshape(1, D), Wr1)
    (agg2,) = _sc_agg(hb, srcb, dstb, zrow)
    (out,) = _dense_plain(agg2, cntp, h, Wl2, bl2.reshape(1, D), Wr2)
    return out


# speedup vs baseline: 1.1361x; 1.0186x over previous
import functools

import jax
import jax.numpy as jnp
from jax import lax
from jax.experimental import pallas as pl
from jax.experimental.pallas import tpu as pltpu
from jax.experimental.pallas import tpu_sc as plsc

N = 10000
E = 320000
D = 128

NC = 2
NS = 16
EPS = E // (NC * NS)
C = 80
CHF = EPS // C
RPS = 640
RLAST = N - 15 * RPS
RCH = 80
CW = 16
NB = 5
GIF = 4
SIF = 1


def _sc_agg_body(do_cnt, *refs):
    if do_cnt:
        (xs, srcb, dstb, zrow, zcnt, orow, agg_o, cnt_o,
         srcv, dstv, rows, zbuf, onesb, aggsh, cntsh, sem) = refs
    else:
        (xs, srcb, dstb, zrow, agg_o,
         srcv, dstv, rows, zbuf, aggsh, sem) = refs

    cid = lax.axis_index("c")
    sid = lax.axis_index("s")

    pltpu.sync_copy(srcb.at[cid, sid], srcv)
    pltpu.sync_copy(dstb.at[cid, sid], dstv)

    pltpu.sync_copy(zrow, zbuf)
    if do_cnt:
        pltpu.sync_copy(orow, onesb)
    r0 = sid * RPS

    @pl.when(sid < NS - 1)
    def _():
        for k in range(RPS // RCH):
            pltpu.sync_copy(zbuf, aggsh.at[pl.ds(r0 + k * RCH, RCH)])
        if do_cnt:
            pltpu.sync_copy(zcnt, cntsh.at[pl.ds(r0, RPS)])

    @pl.when(sid == NS - 1)
    def _():
        for k in range(RLAST // RCH):
            pltpu.sync_copy(zbuf, aggsh.at[pl.ds(r0 + k * RCH, RCH)])
        if do_cnt:
            pltpu.sync_copy(zcnt.at[pl.ds(0, RLAST)], cntsh.at[pl.ds(r0, RLAST)])

    plsc.subcore_barrier()

    semg, sems, semc = sem
    for b in range(GIF):
        pltpu.async_copy(xs.at[srcv.at[b]], rows.at[b], semg)

    def step(gN, carry):
        for b in range(NB):
            g = gN * NB + b
            pltpu.make_async_copy(xs.at[srcv.at[g]], rows.at[b], semg).wait()
            pltpu.async_copy(rows.at[b], aggsh.at[dstv.at[g]], sems, add=True)
            if do_cnt:
                pltpu.async_copy(onesb, cntsh.at[dstv.at[g]], semc, add=True)

            @pl.when(g >= SIF)
            def _():
                pltpu.make_async_copy(rows.at[(b + NB - SIF) % NB],
                                      aggsh.at[dstv.at[g - SIF]], sems).wait()

            @pl.when(g + GIF < CHF)
            def _():
                pltpu.async_copy(xs.at[srcv.at[g + GIF]],
                                 rows.at[(b + GIF) % NB], semg)
        return carry

    lax.fori_loop(0, CHF // NB, step, 0)
    for t in range(SIF):
        g = CHF - SIF + t
        pltpu.make_async_copy(rows.at[g % NB],
                              aggsh.at[dstv.at[g]], sems).wait()
    if do_cnt:
        def drain(i, carry):
            pltpu.make_async_copy(onesb, cntsh.at[dstv.at[i]], semc).wait()
            return carry
        lax.fori_loop(0, CHF, drain, 0)

    plsc.subcore_barrier()

    @pl.when(sid < NS - 1)
    def _():
        pltpu.sync_copy(aggsh.at[pl.ds(r0, RPS)],
                        agg_o.at[cid, pl.ds(r0, RPS)])
        if do_cnt:
            pltpu.sync_copy(cntsh.at[pl.ds(r0, RPS)],
                            cnt_o.at[cid, pl.ds(r0, RPS)])

    @pl.when(sid == NS - 1)
    def _():
        pltpu.sync_copy(aggsh.at[pl.ds(r0, RLAST)],
                        agg_o.at[cid, pl.ds(r0, RLAST)])
        if do_cnt:
            pltpu.sync_copy(cntsh.at[pl.ds(r0, RLAST)],
                            cnt_o.at[cid, pl.ds(r0, RLAST)])


def _make_sc_agg(do_cnt):
    mesh = plsc.VectorSubcoreMesh(core_axis_name="c", subcore_axis_name="s",
                                  num_cores=NC, num_subcores=NS)
    out_type = [jax.ShapeDtypeStruct((NC, N, D), jnp.bfloat16)]
    scratch = [
        pltpu.VMEM((CHF, C), jnp.int32),
        pltpu.VMEM((CHF, C), jnp.int32),
        pltpu.VMEM((NB, C, D), jnp.bfloat16),
        pltpu.VMEM((RCH, D), jnp.bfloat16),
    ]
    if do_cnt:
        out_type.append(jax.ShapeDtypeStruct((NC, N, CW), jnp.float32))
        scratch.append(pltpu.VMEM((C, CW), jnp.float32))
    scratch.append(pltpu.VMEM_SHARED((N, D), jnp.bfloat16))
    if do_cnt:
        scratch.append(pltpu.VMEM_SHARED((N, CW), jnp.float32))
    scratch.append((pltpu.SemaphoreType.DMA, pltpu.SemaphoreType.DMA,
                    pltpu.SemaphoreType.DMA))
    return pl.kernel(functools.partial(_sc_agg_body, do_cnt),
                     out_type=out_type, mesh=mesh, scratch_types=scratch,
                     compiler_params=pltpu.CompilerParams(
                         use_tc_tiling_on_sc=False))


_sc_agg_cnt = _make_sc_agg(True)
_sc_agg = _make_sc_agg(False)

BN = 1000


def _dense_body(relu, agg_ref, cnt_ref, h_ref, wl_ref, bl_ref, wr_ref, o_ref,
                ob_ref=None):
    cnt = cnt_ref[0, :, 0] + cnt_ref[1, :, 0]
    agg = agg_ref[0].astype(jnp.float32) + agg_ref[1].astype(jnp.float32)
    mean = agg / jnp.maximum(cnt, 1.0)[:, None]
    y = jnp.dot(mean, wl_ref[...], preferred_element_type=jnp.float32)
    y = y + jnp.dot(h_ref[...], wr_ref[...], preferred_element_type=jnp.float32)
    y = y + h_ref[...] + bl_ref[...]
    if relu:
        y = jnp.maximum(y, 0.0)
    o_ref[...] = y
    if ob_ref is not None:
        ob_ref[...] = y.astype(jnp.bfloat16)


def _make_dense(relu, emit_bf16):
    out_specs = [pl.BlockSpec((BN, D), lambda i: (i, 0))]
    out_shape = [jax.ShapeDtypeStruct((N, D), jnp.float32)]
    if emit_bf16:
        out_specs.append(pl.BlockSpec((BN, D), lambda i: (i, 0)))
        out_shape.append(jax.ShapeDtypeStruct((N, D), jnp.bfloat16))
    return pl.pallas_call(
        functools.partial(_dense_body, relu),
        grid=(N // BN,),
        in_specs=[
            pl.BlockSpec((NC, BN, D), lambda i: (0, i, 0)),
            pl.BlockSpec((NC, BN, CW), lambda i: (0, i, 0)),
            pl.BlockSpec((BN, D), lambda i: (i, 0)),
            pl.BlockSpec((D, D), lambda i: (0, 0)),
            pl.BlockSpec((1, D), lambda i: (0, 0)),
            pl.BlockSpec((D, D), lambda i: (0, 0)),
        ],
        out_specs=out_specs,
        out_shape=out_shape,
    )


_dense_relu = _make_dense(True, True)
_dense_plain = _make_dense(False, False)


def kernel(x, edge_index, Wl1, bl1, Wr1, Wl2, bl2, Wr2):
    srcb = edge_index[0].reshape(NC, NS, CHF, C)
    dstb = edge_index[1].reshape(NC, NS, CHF, C)
    zrow = jnp.zeros((RCH, D), jnp.bfloat16)
    zcnt = jnp.zeros((RPS, CW), jnp.float32)
    orow = jnp.ones((C, CW), jnp.float32)

    xb = x.astype(jnp.bfloat16)
    agg1, cntp = _sc_agg_cnt(xb, srcb, dstb, zrow, zcnt, orow)
    h, hb = _dense_relu(agg1, cntp, x, Wl1, bl1.reshape(1, D), Wr1)
    (agg2,) = _sc_agg(hb, srcb, dstb, zrow)
    (out,) = _dense_plain(agg2, cntp, h, Wl2, bl2.reshape(1, D), Wr2)
    return out
